# Initial kernel scaffold; baseline (speedup 1.0000x reference)
import jax
import jax.numpy as jnp
from jax.experimental import pallas as pl


def kernel(x, edge_index, W1, b1, W2):
    raise NotImplementedError("write your pallas kernel here")


# speedup vs baseline: 2.0288x; 2.0288x over previous
import jax
import jax.numpy as jnp
from jax import lax
from jax.experimental import pallas as pl
from jax.experimental.pallas import tpu as pltpu
from jax.experimental.pallas import tpu_sc as plsc

_N = 10000
_D = 256
_E = 160000
_H = 128
_NS = 16
_NC = 2
_K = 128
_CHUNKS = 80
_EPT = _K * _CHUNKS
_EPAD = _EPT * _NS
_ROWS_SH = 10112
_RPT = _ROWS_SH // _NS
_TRASH = _N
_MBLK = 1000


def _matmul_body(x_ref, w1_ref, w2_ref, b1_ref,
                 x2a_ref, x2b_ref, da_ref, db_ref):
    xb = x_ref[...]
    w1 = w1_ref[...]
    w2 = w2_ref[...]
    dn = (((1,), (1,)), ((), ()))
    x2 = lax.dot_general(xb, w2, dn, preferred_element_type=jnp.float32)
    d = lax.dot_general(xb, w1 - w2, dn, preferred_element_type=jnp.float32)
    d = d + b1_ref[...]
    x2a_ref[...] = x2[:, :_H]
    x2b_ref[...] = x2[:, _H:]
    da_ref[...] = d[:, :_H]
    db_ref[...] = d[:, _H:]


def _matmuls(x, W1, b1, W2):
    f32 = jnp.float32
    half = pl.BlockSpec((_MBLK, _H), lambda i: (i, 0))
    return pl.pallas_call(
        _matmul_body,
        grid=(_N // _MBLK,),
        in_specs=[
            pl.BlockSpec((_MBLK, _D), lambda i: (i, 0)),
            pl.BlockSpec((_D, _D), lambda i: (0, 0)),
            pl.BlockSpec((_D, _D), lambda i: (0, 0)),
            pl.BlockSpec((1, _D), lambda i: (0, 0)),
        ],
        out_specs=[half, half, half, half],
        out_shape=[jax.ShapeDtypeStruct((_N, _H), f32)] * 4,
    )(x, W1, W2, b1.reshape(1, _D))


def _sc_body(x2a, x2b, da, db, row3d, rowg3d, col3d, z128,
             acc_out,
             acc_sh, row_v, rowg_v, col_v, gbuf, dbuf, sem, semd):
    c = lax.axis_index("c")
    s = lax.axis_index("s")
    base = s * _RPT

    pltpu.sync_copy(z128.at[pl.ds(base, _RPT)], acc_sh.at[pl.ds(base, _RPT)])
    plsc.subcore_barrier()

    def edge_phase(srcx, srcd):
        def grp_body(g, carry):
            pltpu.sync_copy(row3d.at[s, g], row_v)
            pltpu.sync_copy(rowg3d.at[s, g], rowg_v)
            pltpu.sync_copy(col3d.at[s, g], col_v)
            cpx = pltpu.async_copy(srcx.at[col_v], gbuf, sem)
            cpd = pltpu.async_copy(srcd.at[rowg_v], dbuf, semd)
            cpx.wait()
            pltpu.sync_copy(gbuf, acc_sh.at[row_v], add=True)
            cpd.wait()
            pltpu.sync_copy(dbuf, acc_sh.at[row_v], add=True)
            return carry

        lax.fori_loop(0, _CHUNKS, grp_body, 0)

    @pl.when(c == 0)
    def _():
        edge_phase(x2a, da)

    @pl.when(c == 1)
    def _():
        edge_phase(x2b, db)

    plsc.subcore_barrier()

    def writeback(nrows):
        @pl.when(c == 0)
        def _():
            pltpu.sync_copy(acc_sh.at[pl.ds(base, nrows)],
                            acc_out.at[0, pl.ds(base, nrows)])

        @pl.when(c == 1)
        def _():
            pltpu.sync_copy(acc_sh.at[pl.ds(base, nrows)],
                            acc_out.at[1, pl.ds(base, nrows)])

    @pl.when(s < _NS - 1)
    def _():
        writeback(_RPT)

    @pl.when(s == _NS - 1)
    def _():
        writeback(_N - (_NS - 1) * _RPT)


def _sc_scatter(x2a, x2b, da, db, row3d, rowg3d, col3d):
    f32 = jnp.float32
    mesh = plsc.VectorSubcoreMesh(core_axis_name="c", subcore_axis_name="s")
    kern = pl.kernel(
        _sc_body,
        out_type=[
            jax.ShapeDtypeStruct((_NC, _N, _H), f32),
        ],
        mesh=mesh,
        scratch_types=[
            pltpu.VMEM_SHARED((_ROWS_SH, _H), f32),
            pltpu.VMEM((_K,), jnp.int32),
            pltpu.VMEM((_K,), jnp.int32),
            pltpu.VMEM((_K,), jnp.int32),
            pltpu.VMEM((_K, _H), f32),
            pltpu.VMEM((_K, _H), f32),
            pltpu.SemaphoreType.DMA,
            pltpu.SemaphoreType.DMA,
        ],
    )
    z128 = jnp.zeros((_ROWS_SH, _H), f32)
    return kern(x2a, x2b, da, db, row3d, rowg3d, col3d, z128)[0]


def _combine_body(acc_ref, o_ref):
    o_ref[...] = acc_ref[0]


def _combine(acc):
    return pl.pallas_call(
        _combine_body,
        grid=(_N // _MBLK, _NC),
        in_specs=[
            pl.BlockSpec((1, _MBLK, _H), lambda i, h: (h, i, 0)),
        ],
        out_specs=pl.BlockSpec((_MBLK, _H), lambda i, h: (i, h)),
        out_shape=jax.ShapeDtypeStruct((_N, _D), jnp.float32),
    )(acc)


def kernel(x, edge_index, W1, b1, W2):
    x2a, x2b, da, db = _matmuls(x, W1, b1, W2)
    row = edge_index[0].astype(jnp.int32)
    col = edge_index[1].astype(jnp.int32)
    pad = _EPAD - _E
    zpad = jnp.zeros((pad,), jnp.int32)
    row3d = jnp.concatenate(
        [row, jnp.full((pad,), _TRASH, jnp.int32)]).reshape(_NS, _CHUNKS, _K)
    rowg3d = jnp.concatenate([row, zpad]).reshape(_NS, _CHUNKS, _K)
    col3d = jnp.concatenate([col, zpad]).reshape(_NS, _CHUNKS, _K)
    acc = _sc_scatter(x2a, x2b, da, db, row3d, rowg3d, col3d)
    return _combine(acc)


# speedup vs baseline: 2.0832x; 1.0268x over previous
import jax
import jax.numpy as jnp
from jax import lax
from jax.experimental import pallas as pl
from jax.experimental.pallas import tpu as pltpu
from jax.experimental.pallas import tpu_sc as plsc

_N = 10000
_D = 256
_E = 160000
_H = 128
_NS = 16
_NC = 2
_K = 128
_CHUNKS = 80
_EPT = _K * _CHUNKS
_EPAD = _EPT * _NS
_ROWS_SH = 10112
_RPT = _ROWS_SH // _NS
_TRASH = _N
_MBLK = 1000


def _matmul_body(x_ref, w1_ref, w2_ref, b1_ref,
                 x2a_ref, x2b_ref, da_ref, db_ref):
    xb = x_ref[...]
    w1 = w1_ref[...]
    w2 = w2_ref[...]
    dn = (((1,), (1,)), ((), ()))
    x2 = lax.dot_general(xb, w2, dn, preferred_element_type=jnp.float32)
    d = lax.dot_general(xb, w1 - w2, dn, preferred_element_type=jnp.float32)
    d = d + b1_ref[...]
    x2a_ref[...] = x2[:, :_H]
    x2b_ref[...] = x2[:, _H:]
    da_ref[...] = d[:, :_H]
    db_ref[...] = d[:, _H:]


def _matmuls(x, W1, b1, W2):
    f32 = jnp.float32
    half = pl.BlockSpec((_MBLK, _H), lambda i: (i, 0))
    return pl.pallas_call(
        _matmul_body,
        grid=(_N // _MBLK,),
        in_specs=[
            pl.BlockSpec((_MBLK, _D), lambda i: (i, 0)),
            pl.BlockSpec((_D, _D), lambda i: (0, 0)),
            pl.BlockSpec((_D, _D), lambda i: (0, 0)),
            pl.BlockSpec((1, _D), lambda i: (0, 0)),
        ],
        out_specs=[half, half, half, half],
        out_shape=[jax.ShapeDtypeStruct((_N, _H), f32)] * 4,
    )(x, W1, W2, b1.reshape(1, _D))


def _sc_body(x2a, x2b, da, db, row3d, rowg3d, col3d, z128,
             acc_out,
             acc_sh, row_v, rowg_v, col_v, gbuf, dbuf, sem, semd, ssx, ssd):
    c = lax.axis_index("c")
    s = lax.axis_index("s")
    base = s * _RPT

    pltpu.sync_copy(z128.at[pl.ds(base, _RPT)], acc_sh.at[pl.ds(base, _RPT)])
    plsc.subcore_barrier()

    def edge_phase(srcx, srcd):
        def grp_body(g, carry):
            pltpu.sync_copy(row3d.at[s, g], row_v)
            pltpu.sync_copy(rowg3d.at[s, g], rowg_v)
            pltpu.sync_copy(col3d.at[s, g], col_v)
            cpx = pltpu.async_copy(srcx.at[col_v], gbuf, sem)
            cpd = pltpu.async_copy(srcd.at[rowg_v], dbuf, semd)
            cpx.wait()
            cpd.wait()
            c1 = pltpu.async_copy(gbuf, acc_sh.at[row_v], ssx, add=True)
            c2 = pltpu.async_copy(dbuf, acc_sh.at[row_v], ssd, add=True)
            c1.wait()
            c2.wait()
            return carry

        lax.fori_loop(0, _CHUNKS, grp_body, 0)

    @pl.when(c == 0)
    def _():
        edge_phase(x2a, da)

    @pl.when(c == 1)
    def _():
        edge_phase(x2b, db)

    plsc.subcore_barrier()

    def writeback(nrows):
        @pl.when(c == 0)
        def _():
            pltpu.sync_copy(acc_sh.at[pl.ds(base, nrows)],
                            acc_out.at[pl.ds(base, nrows), pl.ds(0, _H)])

        @pl.when(c == 1)
        def _():
            pltpu.sync_copy(acc_sh.at[pl.ds(base, nrows)],
                            acc_out.at[pl.ds(base, nrows), pl.ds(_H, _H)])

    @pl.when(s < _NS - 1)
    def _():
        writeback(_RPT)

    @pl.when(s == _NS - 1)
    def _():
        writeback(_N - (_NS - 1) * _RPT)


def _sc_scatter(x2a, x2b, da, db, row3d, rowg3d, col3d):
    f32 = jnp.float32
    mesh = plsc.VectorSubcoreMesh(core_axis_name="c", subcore_axis_name="s")
    kern = pl.kernel(
        _sc_body,
        out_type=[
            jax.ShapeDtypeStruct((_N, _D), f32),
        ],
        mesh=mesh,
        scratch_types=[
            pltpu.VMEM_SHARED((_ROWS_SH, _H), f32),
            pltpu.VMEM((_K,), jnp.int32),
            pltpu.VMEM((_K,), jnp.int32),
            pltpu.VMEM((_K,), jnp.int32),
            pltpu.VMEM((_K, _H), f32),
            pltpu.VMEM((_K, _H), f32),
            pltpu.SemaphoreType.DMA,
            pltpu.SemaphoreType.DMA,
            pltpu.SemaphoreType.DMA,
            pltpu.SemaphoreType.DMA,
        ],
    )
    z128 = jnp.zeros((_ROWS_SH, _H), f32)
    return kern(x2a, x2b, da, db, row3d, rowg3d, col3d, z128)[0]


def kernel(x, edge_index, W1, b1, W2):
    x2a, x2b, da, db = _matmuls(x, W1, b1, W2)
    row = edge_index[0].astype(jnp.int32)
    col = edge_index[1].astype(jnp.int32)
    pad = _EPAD - _E
    zpad = jnp.zeros((pad,), jnp.int32)
    row3d = jnp.concatenate(
        [row, jnp.full((pad,), _TRASH, jnp.int32)]).reshape(_NS, _CHUNKS, _K)
    rowg3d = jnp.concatenate([row, zpad]).reshape(_NS, _CHUNKS, _K)
    col3d = jnp.concatenate([col, zpad]).reshape(_NS, _CHUNKS, _K)
    return _sc_scatter(x2a, x2b, da, db, row3d, rowg3d, col3d)


# speedup vs baseline: 2.8449x; 1.3656x over previous
import jax
import jax.numpy as jnp
from jax import lax
from jax.experimental import pallas as pl
from jax.experimental.pallas import tpu as pltpu
from jax.experimental.pallas import tpu_sc as plsc

_N = 10000
_D = 256
_E = 160000
_H = 128
_NS = 16
_NC = 2
_K = 64
_CHUNKS = 160
_CPG = 16
_NG = _CHUNKS // _CPG
_EPT = _K * _CHUNKS
_EPAD = _EPT * _NS
_ROWS_SH = 10112
_RPT = _ROWS_SH // _NS
_TRASH = _N
_MBLK = 1000


def _matmul_body(x_ref, w1_ref, w2_ref, b1_ref,
                 x2a_ref, x2b_ref, da_ref, db_ref):
    xb = x_ref[...]
    w1 = w1_ref[...]
    w2 = w2_ref[...]
    dn = (((1,), (1,)), ((), ()))
    x2 = lax.dot_general(xb, w2, dn, preferred_element_type=jnp.float32)
    d = lax.dot_general(xb, w1 - w2, dn, preferred_element_type=jnp.float32)
    d = d + b1_ref[...]
    x2a_ref[...] = x2[:, :_H]
    x2b_ref[...] = x2[:, _H:]
    da_ref[...] = d[:, :_H]
    db_ref[...] = d[:, _H:]


def _matmuls(x, W1, b1, W2):
    f32 = jnp.float32
    half = pl.BlockSpec((_MBLK, _H), lambda i: (i, 0))
    return pl.pallas_call(
        _matmul_body,
        grid=(_N // _MBLK,),
        in_specs=[
            pl.BlockSpec((_MBLK, _D), lambda i: (i, 0)),
            pl.BlockSpec((_D, _D), lambda i: (0, 0)),
            pl.BlockSpec((_D, _D), lambda i: (0, 0)),
            pl.BlockSpec((1, _D), lambda i: (0, 0)),
        ],
        out_specs=[half, half, half, half],
        out_shape=[jax.ShapeDtypeStruct((_N, _H), f32)] * 4,
    )(x, W1, W2, b1.reshape(1, _D))


def _sc_body(x2a, x2b, da, db, row4d, rowg4d, col4d, z128,
             acc_out,
             acc_sh, row_g, rowg_g, col_g,
             gbuf0, gbuf1, dbuf0, dbuf1,
             sem, semd, ssx0, ssx1, ssd0, ssd1):
    c = lax.axis_index("c")
    s = lax.axis_index("s")
    base = s * _RPT

    pltpu.sync_copy(z128.at[pl.ds(base, _RPT)], acc_sh.at[pl.ds(base, _RPT)])
    plsc.subcore_barrier()

    gb = (gbuf0, gbuf1)
    db_ = (dbuf0, dbuf1)
    sx = (ssx0, ssx1)
    sd = (ssd0, ssd1)

    def edge_phase(srcx, srcd):
        def grp_body(gi, carry):
            pltpu.sync_copy(row4d.at[s, gi], row_g)
            pltpu.sync_copy(rowg4d.at[s, gi], rowg_g)
            pltpu.sync_copy(col4d.at[s, gi], col_g)
            prime = (gi == 0)

            def pair_body(pj, carry2):
                for b in range(2):
                    j = pj * 2 + b

                    @pl.when(jnp.logical_not(prime & (pj == 0)))
                    def _():
                        pltpu.make_async_copy(
                            gb[b], acc_sh.at[row_g.at[j]], sx[b]).wait()
                        pltpu.make_async_copy(
                            db_[b], acc_sh.at[row_g.at[j]], sd[b]).wait()

                    cpx = pltpu.async_copy(srcx.at[col_g.at[j]], gb[b], sem)
                    cpd = pltpu.async_copy(srcd.at[rowg_g.at[j]], db_[b], semd)
                    cpx.wait()
                    cpd.wait()
                    pltpu.async_copy(gb[b], acc_sh.at[row_g.at[j]], sx[b],
                                     add=True)
                    pltpu.async_copy(db_[b], acc_sh.at[row_g.at[j]], sd[b],
                                     add=True)
                return carry2

            lax.fori_loop(0, _CPG // 2, pair_body, 0)
            return carry

        lax.fori_loop(0, _NG, grp_body, 0)
        for b in range(2):
            pltpu.make_async_copy(gb[b], acc_sh.at[row_g.at[0]], sx[b]).wait()
            pltpu.make_async_copy(db_[b], acc_sh.at[row_g.at[0]], sd[b]).wait()

    @pl.when(c == 0)
    def _():
        edge_phase(x2a, da)

    @pl.when(c == 1)
    def _():
        edge_phase(x2b, db)

    plsc.subcore_barrier()

    def writeback(nrows):
        @pl.when(c == 0)
        def _():
            pltpu.sync_copy(acc_sh.at[pl.ds(base, nrows)],
                            acc_out.at[pl.ds(base, nrows), pl.ds(0, _H)])

        @pl.when(c == 1)
        def _():
            pltpu.sync_copy(acc_sh.at[pl.ds(base, nrows)],
                            acc_out.at[pl.ds(base, nrows), pl.ds(_H, _H)])

    @pl.when(s < _NS - 1)
    def _():
        writeback(_RPT)

    @pl.when(s == _NS - 1)
    def _():
        writeback(_N - (_NS - 1) * _RPT)


def _sc_scatter(x2a, x2b, da, db, row4d, rowg4d, col4d):
    f32 = jnp.float32
    mesh = plsc.VectorSubcoreMesh(core_axis_name="c", subcore_axis_name="s")
    kern = pl.kernel(
        _sc_body,
        out_type=[
            jax.ShapeDtypeStruct((_N, _D), f32),
        ],
        mesh=mesh,
        scratch_types=[
            pltpu.VMEM_SHARED((_ROWS_SH, _H), f32),
            pltpu.VMEM((_CPG, _K), jnp.int32),
            pltpu.VMEM((_CPG, _K), jnp.int32),
            pltpu.VMEM((_CPG, _K), jnp.int32),
            pltpu.VMEM((_K, _H), f32),
            pltpu.VMEM((_K, _H), f32),
            pltpu.VMEM((_K, _H), f32),
            pltpu.VMEM((_K, _H), f32),
            pltpu.SemaphoreType.DMA,
            pltpu.SemaphoreType.DMA,
            pltpu.SemaphoreType.DMA,
            pltpu.SemaphoreType.DMA,
            pltpu.SemaphoreType.DMA,
            pltpu.SemaphoreType.DMA,
        ],
    )
    z128 = jnp.zeros((_ROWS_SH, _H), f32)
    return kern(x2a, x2b, da, db, row4d, rowg4d, col4d, z128)[0]


def kernel(x, edge_index, W1, b1, W2):
    x2a, x2b, da, db = _matmuls(x, W1, b1, W2)
    row = edge_index[0].astype(jnp.int32)
    col = edge_index[1].astype(jnp.int32)
    pad = _EPAD - _E
    zpad = jnp.zeros((pad,), jnp.int32)
    shp = (_NS, _NG, _CPG, _K)
    row4d = jnp.concatenate(
        [row, jnp.full((pad,), _TRASH, jnp.int32)]).reshape(shp)
    rowg4d = jnp.concatenate([row, zpad]).reshape(shp)
    col4d = jnp.concatenate([col, zpad]).reshape(shp)
    return _sc_scatter(x2a, x2b, da, db, row4d, rowg4d, col4d)
